# baseline (speedup 1.0000x reference)
import functools

import jax
import jax.numpy as jnp
from jax import lax
from jax.experimental import pallas as pl
from jax.experimental.pallas import tpu as pltpu
from jax.experimental.pallas import tpu_sc as plsc

B, L, D = 16384, 200, 64
SCALE = 8.0

NC, NS = 2, 16
NW = NC * NS
B_PER_W = B // NW
UB = 128
UNITS = B_PER_W // UB
L_SLAB = 10
SLABS = L // L_SLAB

_PRE_ROWS = 1000


def _prescale_body(table_ref, out_ref):
    i = pl.program_id(0)
    row = lax.broadcasted_iota(jnp.int32, table_ref.shape, 0) + i * _PRE_ROWS
    out_ref[...] = jnp.where(row == 0, 0.0, table_ref[...] * SCALE)


def _prescale(table):
    v, d = table.shape
    return pl.pallas_call(
        _prescale_body,
        grid=(v // _PRE_ROWS,),
        in_specs=[pl.BlockSpec((_PRE_ROWS, d), lambda i: (i, 0))],
        out_specs=pl.BlockSpec((_PRE_ROWS, d), lambda i: (i, 0)),
        out_shape=jax.ShapeDtypeStruct((v, d), jnp.float32),
    )(table)


_mesh = plsc.VectorSubcoreMesh(core_axis_name="c", subcore_axis_name="s")


@functools.partial(
    pl.kernel,
    mesh=_mesh,
    out_type=jax.ShapeDtypeStruct((L, D // 8, B // 128, 8, 128), jnp.float32),
    scratch_types=[
        [pltpu.VMEM((L_SLAB, B_PER_W), jnp.int32) for _ in range(2)],
        [pltpu.VMEM((UB, D), jnp.float32) for _ in range(2)]
        + [pltpu.VMEM((UB, D + 1), jnp.float32) for _ in range(2)],
        [pltpu.VMEM((D, 128), jnp.float32) for _ in range(2)],
        [pltpu.SemaphoreType.DMA for _ in range(2)],
        [pltpu.SemaphoreType.DMA for _ in range(2)],
        pltpu.SemaphoreType.DMA,
    ],
    compiler_params=pltpu.CompilerParams(use_tc_tiling_on_sc=False, needs_layout_passes=False, disable_bounds_checks=True),
)
def _tgather(table_hbm, idx_hbm, out_hbm, idx_bufs, gbufs, tbufs, gsems, ssems, isem):
    wid = lax.axis_index("s") * NC + lax.axis_index("c")
    b0 = wid * B_PER_W
    iota16 = lax.broadcasted_iota(jnp.int32, (16,), 0)
    rowidx = [iota16 + 16 * c for c in range(8)]

    def repitch_unit(gbuf, gbuf68):
        @plsc.parallel_loop(0, UB, unroll=8)
        def b_body(b):
            for k in range(D // 16):
                gbuf68[b, pl.ds(16 * k, 16)] = gbuf[b, pl.ds(16 * k, 16)]

    def transpose_unit(gbuf, tbuf):
        @plsc.parallel_loop(0, D // 8, unroll=4)
        def t_body(t):
            for r in range(8):
                d = 8 * t + r
                dcol = jnp.full((16,), d, jnp.int32)
                for c in range(8):
                    v = plsc.load_gather(gbuf, [rowidx[c], dcol])
                    tbuf[d, pl.ds(16 * c, 16)] = v

    def fire_idx(s, ib):
        pltpu.async_copy(
            idx_hbm.at[pl.ds(s * L_SLAB, L_SLAB), pl.ds(b0, B_PER_W)],
            idx_bufs[ib],
            isem,
        )

    def wait_idx(ib):
        pltpu.make_async_copy(
            idx_hbm.at[pl.ds(0, L_SLAB), pl.ds(b0, B_PER_W)],
            idx_bufs[ib],
            isem,
        ).wait()

    fire_idx(0, 0)
    wait_idx(0)

    def process_slab(s, half):
        @pl.when(s + 1 < SLABS)
        def _():
            fire_idx(s + 1, (half + 1) % 2)

        def l_body(lrel, carry):
            l = s * L_SLAB + lrel

            def fire_gather(u):
                return pltpu.async_copy(
                    table_hbm.at[idx_bufs[half].at[lrel, pl.ds(u * UB, UB)]],
                    gbufs[u % 2],
                    gsems[u % 2],
                )

            def drain_stores(ub, l_):
                for t in range(D // 8):
                    pltpu.make_async_copy(
                        tbufs[ub].at[pl.ds(8 * t, 8)],
                        out_hbm.at[l_, t, 4 * wid + ub],
                        ssems[ub],
                    ).wait()

            cp = fire_gather(0)
            for u in range(UNITS):
                nxt = fire_gather(u + 1) if u + 1 < UNITS else None
                cp.wait()
                cp = nxt
                if u < 2:
                    @pl.when(l > 0)
                    def _():
                        drain_stores(u % 2, l)
                else:
                    drain_stores(u % 2, l)
                repitch_unit(gbufs[u % 2], gbufs[2 + u % 2])
                transpose_unit(gbufs[2 + u % 2], tbufs[u % 2])
                for t in range(D // 8):
                    pltpu.async_copy(
                        tbufs[u % 2].at[pl.ds(8 * t, 8)],
                        out_hbm.at[l, t, 4 * wid + u],
                        ssems[u % 2],
                    )
            return carry

        lax.fori_loop(0, L_SLAB, l_body, 0)

        @pl.when(s + 1 < SLABS)
        def _():
            wait_idx((half + 1) % 2)

    def pair_body(p, carry):
        for half in range(2):
            process_slab(2 * p + half, half)
        return carry

    lax.fori_loop(0, SLABS // 2, pair_body, 0)

    for ub in range(2):
        for t in range(D // 8):
            pltpu.make_async_copy(
                tbufs[ub].at[pl.ds(8 * t, 8)],
                out_hbm.at[L - 1, t, 4 * wid + 2 + ub],
                ssems[ub],
            ).wait()


def kernel(inputs, table):
    scaled = _prescale(table.astype(jnp.float32))
    idx_t = inputs.T.astype(jnp.int32)
    out5 = _tgather(scaled, idx_t)
    return out5.transpose(2, 4, 0, 1, 3).reshape(B, L, D)


# speedup vs baseline: 1.0088x; 1.0088x over previous
import functools

import jax
import jax.numpy as jnp
from jax import lax
from jax.experimental import pallas as pl
from jax.experimental.pallas import tpu as pltpu
from jax.experimental.pallas import tpu_sc as plsc

B, L, D = 16384, 200, 64
SCALE = 8.0

NC, NS = 2, 16
NW = NC * NS
B_PER_W = B // NW
UB = 128
UNITS = B_PER_W // UB
L_SLAB = 10
SLABS = L // L_SLAB

_PRE_ROWS = 1000


def _prescale_body(table_ref, out_ref):
    i = pl.program_id(0)
    row = lax.broadcasted_iota(jnp.int32, table_ref.shape, 0) + i * _PRE_ROWS
    out_ref[...] = jnp.where(row == 0, 0.0, table_ref[...] * SCALE)


def _prescale(table):
    v, d = table.shape
    return pl.pallas_call(
        _prescale_body,
        grid=(v // _PRE_ROWS,),
        in_specs=[pl.BlockSpec((_PRE_ROWS, d), lambda i: (i, 0))],
        out_specs=pl.BlockSpec((_PRE_ROWS, d), lambda i: (i, 0)),
        out_shape=jax.ShapeDtypeStruct((v, d), jnp.float32),
    )(table)


_mesh = plsc.VectorSubcoreMesh(core_axis_name="c", subcore_axis_name="s")


@functools.partial(
    pl.kernel,
    mesh=_mesh,
    out_type=jax.ShapeDtypeStruct((L, D // 8, B // 128, 8, 128), jnp.float32),
    scratch_types=[
        [pltpu.VMEM((L_SLAB, B_PER_W), jnp.int32) for _ in range(2)],
        [pltpu.VMEM((UB, D), jnp.float32) for _ in range(2)]
        + [pltpu.VMEM((UB, D + 1), jnp.float32) for _ in range(2)],
        [pltpu.VMEM((D, 128), jnp.float32) for _ in range(2)],
        [pltpu.SemaphoreType.DMA for _ in range(2)],
        [pltpu.SemaphoreType.DMA for _ in range(2)],
        pltpu.SemaphoreType.DMA,
    ],
    compiler_params=pltpu.CompilerParams(
        use_tc_tiling_on_sc=False,
        needs_layout_passes=False,
        disable_bounds_checks=True,
    ),
)
def _tgather(table_hbm, idx_hbm, out_hbm, idx_bufs, gbufs, tbufs, gsems, ssems, isem):
    wid = lax.axis_index("s") * NC + lax.axis_index("c")
    b0 = wid * B_PER_W
    iota16 = lax.broadcasted_iota(jnp.int32, (16,), 0)
    rowidx = [iota16 + 16 * c for c in range(8)]

    def repitch_unit(gbuf, gbufp):
        @plsc.parallel_loop(0, UB, unroll=4)
        def b_body(b):
            for k in range(D // 16):
                gbufp[b, pl.ds(16 * k, 16)] = gbuf[b, pl.ds(16 * k, 16)]

    def transpose_unit(gbuf, tbuf):
        @plsc.parallel_loop(0, D // 8, unroll=4)
        def t_body(t):
            for r in range(8):
                d = 8 * t + r
                dcol = jnp.full((16,), d, jnp.int32)
                for c in range(8):
                    v = plsc.load_gather(gbuf, [rowidx[c], dcol])
                    tbuf[d, pl.ds(16 * c, 16)] = v

    def fire_idx(s, ib):
        pltpu.async_copy(
            idx_hbm.at[pl.ds(s * L_SLAB, L_SLAB), pl.ds(b0, B_PER_W)],
            idx_bufs[ib],
            isem,
        )

    def wait_idx(ib):
        pltpu.make_async_copy(
            idx_hbm.at[pl.ds(0, L_SLAB), pl.ds(b0, B_PER_W)],
            idx_bufs[ib],
            isem,
        ).wait()

    fire_idx(0, 0)
    wait_idx(0)

    def process_slab(s, half):
        @pl.when(s + 1 < SLABS)
        def _():
            fire_idx(s + 1, (half + 1) % 2)

        def l_body(lrel, carry):
            l = s * L_SLAB + lrel

            def fire_gather(u):
                return pltpu.async_copy(
                    table_hbm.at[idx_bufs[half].at[lrel, pl.ds(u * UB, UB)]],
                    gbufs[u % 2],
                    gsems[u % 2],
                )

            def drain_stores(ub, l_):
                for t in range(D // 8):
                    pltpu.make_async_copy(
                        tbufs[ub].at[pl.ds(8 * t, 8)],
                        out_hbm.at[l_, t, 4 * wid + ub],
                        ssems[ub],
                    ).wait()

            cp = fire_gather(0)
            for u in range(UNITS):
                nxt = fire_gather(u + 1) if u + 1 < UNITS else None
                cp.wait()
                cp = nxt
                if u < 2:
                    @pl.when(l > 0)
                    def _():
                        drain_stores(u % 2, l)
                else:
                    drain_stores(u % 2, l)
                repitch_unit(gbufs[u % 2], gbufs[2 + u % 2])
                transpose_unit(gbufs[2 + u % 2], tbufs[u % 2])
                for t in range(D // 8):
                    pltpu.async_copy(
                        tbufs[u % 2].at[pl.ds(8 * t, 8)],
                        out_hbm.at[l, t, 4 * wid + u],
                        ssems[u % 2],
                    )
            return carry

        lax.fori_loop(0, L_SLAB, l_body, 0)

        @pl.when(s + 1 < SLABS)
        def _():
            wait_idx((half + 1) % 2)

    def pair_body(p, carry):
        for half in range(2):
            process_slab(2 * p + half, half)
        return carry

    lax.fori_loop(0, SLABS // 2, pair_body, 0)

    for ub in range(2):
        for t in range(D // 8):
            pltpu.make_async_copy(
                tbufs[ub].at[pl.ds(8 * t, 8)],
                out_hbm.at[L - 1, t, 4 * wid + 2 + ub],
                ssems[ub],
            ).wait()


def kernel(inputs, table):
    scaled = _prescale(table.astype(jnp.float32))
    idx_t = inputs.T.astype(jnp.int32)
    out5 = _tgather(scaled, idx_t)
    return out5.transpose(2, 4, 0, 1, 3).reshape(B, L, D)
